# baseline (speedup 1.0000x reference)
import functools

import jax
import jax.numpy as jnp
from jax import lax
from jax.experimental import pallas as pl
from jax.experimental.pallas import tpu as pltpu
from jax.experimental.pallas import tpu_sc as plsc

B, T, D = 1, 2048, 1024
INTER = 512
E = 8
ROUTE_SCALE = 2.5

BT = 1024
BLK = 256
NB = 23
NPAD = NB * BLK
NTILES = 32
SLOTS_PER_TILE = NPAD // NTILES
DP = D // 2
GCH = 48


def _ffn(xbh, g_ref, u_ref, d_ref):
    dn = (((1,), (1,)), ((), ()))
    g = lax.dot_general(xbh, g_ref[0], dn, preferred_element_type=jnp.float32)
    u = lax.dot_general(xbh, u_ref[0], dn, preferred_element_type=jnp.float32)
    h = (g * (1.0 / (1.0 + jnp.exp(-g)))) * u
    return lax.dot_general(h.astype(jnp.bfloat16), d_ref[0], dn,
                           preferred_element_type=jnp.float32)


def _k1_body(x_ref, gate_ref, sg_ref, su_ref, sd_ref,
             shared_ref, aux_ref, i1_ref, i2_ref, w1_ref, w2_ref,
             r1_ref, r2_ref, cnt_ref, xbf_ref, sums_ref, carry_ref):
    t = pl.program_id(0)
    xb = x_ref[:]

    logits = lax.dot_general(
        xb, gate_ref[:], (((1,), (1,)), ((), ())),
        preferred_element_type=jnp.float32) * ROUTE_SCALE
    mx = jnp.max(logits, axis=1, keepdims=True)
    ex = jnp.exp(logits - mx)
    scores = ex / jnp.sum(ex, axis=1, keepdims=True)
    iota8 = lax.broadcasted_iota(jnp.int32, (BT, E), 1)
    m1 = jnp.max(scores, axis=1, keepdims=True)
    i1 = jnp.min(jnp.where(scores == m1, iota8, E), axis=1, keepdims=True)
    masked = jnp.where(iota8 == i1, -jnp.inf, scores)
    m2 = jnp.max(masked, axis=1, keepdims=True)
    i2 = jnp.min(jnp.where(masked == m2, iota8, E), axis=1, keepdims=True)
    s = m1 + m2
    i1_ref[:] = i1
    i2_ref[:] = i2
    w1_ref[:] = m1 / s
    w2_ref[:] = m2 / s

    onehot = ((iota8 == i1) | (iota8 == i2)).astype(jnp.bfloat16)

    @pl.when(t == 0)
    def _():
        sums_ref[:] = jnp.zeros_like(sums_ref)
        carry_ref[:] = jnp.zeros_like(carry_ref)

    rows = lax.broadcasted_iota(jnp.int32, (BT, BT), 0)
    cols = lax.broadcasted_iota(jnp.int32, (BT, BT), 1)
    tri = (cols < rows).astype(jnp.bfloat16)
    excl = lax.dot_general(tri, onehot, (((1,), (0,)), ((), ())),
                           preferred_element_type=jnp.float32)
    excl = excl + carry_ref[0:1, :]
    r1_ref[:] = jnp.sum(jnp.where(iota8 == i1, excl, 0.0), axis=1,
                        keepdims=True)
    r2_ref[:] = jnp.sum(jnp.where(iota8 == i2, excl, 0.0), axis=1,
                        keepdims=True)

    ohf = onehot.astype(jnp.float32)
    carry_ref[:] += jnp.sum(ohf, axis=0, keepdims=True)
    cnt_ref[:] = carry_ref[:]
    sums_ref[0:1, :] += jnp.sum(ohf, axis=0, keepdims=True)
    sums_ref[1:2, :] += jnp.sum(scores, axis=0, keepdims=True)
    aux_ref[:] = (E / (T * T)) * jnp.sum(
        sums_ref[0:1, :] * sums_ref[1:2, :], axis=1, keepdims=True)

    xbc = xb.astype(jnp.bfloat16)
    xbf_ref[:] = xbc
    shared_ref[:] = _ffn(xbc, sg_ref, su_ref, sd_ref)


def _run_k1(flat, gate_w, sg, su, sd):
    return pl.pallas_call(
        _k1_body,
        grid=(T // BT,),
        in_specs=[
            pl.BlockSpec((BT, D), lambda t: (t, 0)),
            pl.BlockSpec((E, D), lambda t: (0, 0)),
            pl.BlockSpec((1, INTER, D), lambda t: (0, 0, 0)),
            pl.BlockSpec((1, INTER, D), lambda t: (0, 0, 0)),
            pl.BlockSpec((1, D, INTER), lambda t: (0, 0, 0)),
        ],
        out_specs=[
            pl.BlockSpec((BT, D), lambda t: (t, 0)),
            pl.BlockSpec((1, 1), lambda t: (0, 0)),
            pl.BlockSpec((BT, 1), lambda t: (t, 0)),
            pl.BlockSpec((BT, 1), lambda t: (t, 0)),
            pl.BlockSpec((BT, 1), lambda t: (t, 0)),
            pl.BlockSpec((BT, 1), lambda t: (t, 0)),
            pl.BlockSpec((BT, 1), lambda t: (t, 0)),
            pl.BlockSpec((BT, 1), lambda t: (t, 0)),
            pl.BlockSpec((1, E), lambda t: (0, 0)),
            pl.BlockSpec((BT, D), lambda t: (t, 0)),
        ],
        out_shape=[
            jax.ShapeDtypeStruct((T, D), jnp.float32),
            jax.ShapeDtypeStruct((1, 1), jnp.float32),
            jax.ShapeDtypeStruct((T, 1), jnp.int32),
            jax.ShapeDtypeStruct((T, 1), jnp.int32),
            jax.ShapeDtypeStruct((T, 1), jnp.float32),
            jax.ShapeDtypeStruct((T, 1), jnp.float32),
            jax.ShapeDtypeStruct((T, 1), jnp.float32),
            jax.ShapeDtypeStruct((T, 1), jnp.float32),
            jax.ShapeDtypeStruct((1, E), jnp.float32),
            jax.ShapeDtypeStruct((T, D), jnp.bfloat16),
        ],
        scratch_shapes=[
            pltpu.VMEM((2, E), jnp.float32),
            pltpu.VMEM((1, E), jnp.float32),
        ],
        compiler_params=pltpu.CompilerParams(
            dimension_semantics=("arbitrary",)),
    )(flat, gate_w, sg, su, sd)


def _k2_body(xpk_hbm, i1h, i2h, r1h, r2h, w1h, w2h, cnth,
             xs_out, ws_out, s1_out, s2_out, be_out,
             i1v, i2v, r1v, r2v, w1v, w2v, cntv, bsr, bsb, nbr,
             tokv, wsv, s1v, s2v, besv, rows0, rows1, sem0, sem1):
    cid = lax.axis_index("c")
    sid = lax.axis_index("s")
    wid = sid * 2 + cid
    base = wid * SLOTS_PER_TILE

    pltpu.sync_copy(i1h, i1v)
    pltpu.sync_copy(i2h, i2v)
    pltpu.sync_copy(r1h, r1v)
    pltpu.sync_copy(r2h, r2v)
    pltpu.sync_copy(w1h, w1v)
    pltpu.sync_copy(w2h, w2v)
    pltpu.sync_copy(cnth, cntv)

    cnt = cntv[...]
    lane = lax.iota(jnp.int32, 16)
    nbr[...] = (cnt + (BLK - 1)) >> 8
    startb = jnp.zeros((16,), jnp.int32)
    for j in range(E):
        nbj = plsc.load_gather(nbr, [jnp.full((16,), j, jnp.int32)])
        startb = startb + jnp.where(lane > j, nbj, 0)
    bsr[...] = startb * BLK
    bsb[...] = startb

    for off in [i * 16 for i in range(SLOTS_PER_TILE // 16)] + [
            SLOTS_PER_TILE - 16]:
        tokv[pl.ds(base + off, 16)] = jnp.zeros((16,), jnp.int32)

    def _scatter(i, c):
        sl = pl.ds(i * 16, 16)
        toks = lane + i * 16
        e1 = i1v[sl]
        s1 = plsc.load_gather(bsr, [e1]) + r1v[sl]
        plsc.store_scatter(tokv, [s1], toks)
        plsc.store_scatter(wsv, [s1], w1v[sl])
        s1v[sl] = s1
        e2 = i2v[sl]
        s2 = plsc.load_gather(bsr, [e2]) + r2v[sl]
        plsc.store_scatter(tokv, [s2], toks)
        plsc.store_scatter(wsv, [s2], w2v[sl])
        s2v[sl] = s2
        return c
    lax.fori_loop(0, T // 16, _scatter, 0)

    for v in range(2):
        bidx = lane + v * 16
        acc = jnp.full((16,), -1, jnp.int32)
        for e in range(E):
            se = plsc.load_gather(bsb, [jnp.full((16,), e, jnp.int32)])
            acc = acc + jnp.where(bidx >= se, 1, 0)
        besv[pl.ds(v * 16, 16)] = jnp.minimum(acc, E - 1)

    @pl.when(wid == 1)
    def _():
        pltpu.sync_copy(wsv, ws_out)

    @pl.when(wid == 2)
    def _():
        pltpu.sync_copy(s1v, s1_out)

    @pl.when(wid == 3)
    def _():
        pltpu.sync_copy(s2v, s2_out)

    @pl.when(wid == 4)
    def _():
        pltpu.sync_copy(besv, be_out)

    chunks = ((0, 48), (48, 48), (96, 48), (144, 40))
    bufs = (rows0, rows1)
    sems = (sem0, sem1)
    cps = [None, None]
    for c, (off, sz) in enumerate(chunks):
        b = c & 1
        if cps[b] is not None:
            poff, psz = chunks[c - 2]
            cps[b].wait()
            pltpu.sync_copy(bufs[b].at[pl.ds(0, psz)],
                            xs_out.at[pl.ds(base + poff, psz)])
        idx = tokv.at[pl.ds(base + off, sz)]
        cps[b] = pltpu.async_copy(xpk_hbm.at[idx], bufs[b].at[pl.ds(0, sz)],
                                  sems[b])
    for c in (2, 3):
        b = c & 1
        poff, psz = chunks[c]
        cps[b].wait()
        pltpu.sync_copy(bufs[b].at[pl.ds(0, psz)],
                        xs_out.at[pl.ds(base + poff, psz)])


def _run_k2(xpk, i1f, i2f, r1f, r2f, w1f, w2f, cnt16):
    mesh = plsc.VectorSubcoreMesh(core_axis_name="c", subcore_axis_name="s")
    k = functools.partial(
        pl.kernel, mesh=mesh,
        out_type=[
            jax.ShapeDtypeStruct((NPAD, DP), jnp.int32),
            jax.ShapeDtypeStruct((NPAD,), jnp.float32),
            jax.ShapeDtypeStruct((T,), jnp.int32),
            jax.ShapeDtypeStruct((T,), jnp.int32),
            jax.ShapeDtypeStruct((32,), jnp.int32),
        ],
        scratch_types=[
            pltpu.VMEM((T,), jnp.int32),
            pltpu.VMEM((T,), jnp.int32),
            pltpu.VMEM((T,), jnp.int32),
            pltpu.VMEM((T,), jnp.int32),
            pltpu.VMEM((T,), jnp.float32),
            pltpu.VMEM((T,), jnp.float32),
            pltpu.VMEM((16,), jnp.int32),
            pltpu.VMEM((16,), jnp.int32),
            pltpu.VMEM((16,), jnp.int32),
            pltpu.VMEM((16,), jnp.int32),
            pltpu.VMEM((NPAD,), jnp.int32),
            pltpu.VMEM((NPAD,), jnp.float32),
            pltpu.VMEM((T,), jnp.int32),
            pltpu.VMEM((T,), jnp.int32),
            pltpu.VMEM((32,), jnp.int32),
            pltpu.VMEM((GCH, DP), jnp.int32),
            pltpu.VMEM((GCH, DP), jnp.int32),
            pltpu.SemaphoreType.DMA,
            pltpu.SemaphoreType.DMA,
        ],
        compiler_params=pltpu.CompilerParams(needs_layout_passes=False),
    )(_k2_body)
    return k(xpk, i1f, i2f, r1f, r2f, w1f, w2f, cnt16)


def _k3_body(be_ref, xs_ref, w_ref, rg_ref, ru_ref, rd_ref, ys_ref):
    y = _ffn(xs_ref[:], rg_ref, ru_ref, rd_ref)
    ys_ref[:] = w_ref[:] * y


def _run_k3(xs, wslot, be, rg, ru, rd):
    grid_spec = pltpu.PrefetchScalarGridSpec(
        num_scalar_prefetch=1,
        grid=(NB,),
        in_specs=[
            pl.BlockSpec((BLK, D), lambda i, be: (i, 0)),
            pl.BlockSpec((BLK, 1), lambda i, be: (i, 0)),
            pl.BlockSpec((1, INTER, D), lambda i, be: (be[i], 0, 0)),
            pl.BlockSpec((1, INTER, D), lambda i, be: (be[i], 0, 0)),
            pl.BlockSpec((1, D, INTER), lambda i, be: (be[i], 0, 0)),
        ],
        out_specs=pl.BlockSpec((BLK, D), lambda i, be: (i, 0)),
    )
    return pl.pallas_call(
        _k3_body,
        grid_spec=grid_spec,
        out_shape=jax.ShapeDtypeStruct((NPAD, D), jnp.float32),
        compiler_params=pltpu.CompilerParams(
            dimension_semantics=("arbitrary",)),
    )(be, xs, wslot, rg, ru, rd)


def _k4_body(shared_hbm, ys_hbm, s1_hbm, s2_hbm, out_hbm,
             iv1, iv2, acc, sem):
    cid = lax.axis_index("c")
    sid = lax.axis_index("s")
    wid = sid * 2 + cid
    base = wid * (T // NTILES)
    n = T // NTILES

    pltpu.sync_copy(s1_hbm.at[pl.ds(base, n)], iv1)
    pltpu.sync_copy(s2_hbm.at[pl.ds(base, n)], iv2)
    pltpu.sync_copy(shared_hbm.at[pl.ds(base, n)], acc)
    pltpu.async_copy(ys_hbm.at[iv1], acc, sem, add=True).wait()
    pltpu.async_copy(ys_hbm.at[iv2], acc, sem, add=True).wait()
    pltpu.sync_copy(acc, out_hbm.at[pl.ds(base, n)])


def _run_k4(shared, ys, s1, s2):
    mesh = plsc.VectorSubcoreMesh(core_axis_name="c", subcore_axis_name="s")
    k = functools.partial(
        pl.kernel, mesh=mesh,
        out_type=jax.ShapeDtypeStruct((T, D), jnp.float32),
        scratch_types=[
            pltpu.VMEM((T // NTILES,), jnp.int32),
            pltpu.VMEM((T // NTILES,), jnp.int32),
            pltpu.VMEM((T // NTILES, D), jnp.float32),
            pltpu.SemaphoreType.DMA,
        ],
        compiler_params=pltpu.CompilerParams(needs_layout_passes=False),
    )(_k4_body)
    return k(shared, ys, s1, s2)


@jax.jit
def kernel(x, gate_w, shared_gate, shared_up, shared_down,
           routed_gate, routed_up, routed_down):
    flat = x.reshape(T, D)
    bf = jnp.bfloat16

    (shared, aux, i1, i2, w1, w2, r1, r2, cnt, xbf) = _run_k1(
        flat, gate_w, shared_gate.astype(bf), shared_up.astype(bf),
        shared_down.astype(bf))

    i1f = i1.reshape(T)
    i2f = i2.reshape(T)
    r1f = r1.reshape(T).astype(jnp.int32)
    r2f = r2.reshape(T).astype(jnp.int32)
    w1f = w1.reshape(T)
    w2f = w2.reshape(T)
    cnt16 = jnp.zeros((16,), jnp.int32).at[:E].set(
        cnt.reshape(E).astype(jnp.int32))

    xpk = lax.bitcast_convert_type(xbf.reshape(T, DP, 2), jnp.int32)

    xs_pk, wslot, s1, s2, be = _run_k2(
        xpk, i1f, i2f, r1f, r2f, w1f, w2f, cnt16)

    xsb = lax.bitcast_convert_type(xs_pk, bf).reshape(NPAD, D)

    ys = _run_k3(xsb, wslot.reshape(NPAD, 1), be,
                 routed_gate.astype(bf), routed_up.astype(bf),
                 routed_down.astype(bf))

    out = _run_k4(shared, ys, s1, s2)
    return out.reshape(B, T, D), aux[0, 0]


# speedup vs baseline: 1.5810x; 1.5810x over previous
import functools

import jax
import jax.numpy as jnp
from jax import lax
from jax.experimental import pallas as pl
from jax.experimental.pallas import tpu as pltpu
from jax.experimental.pallas import tpu_sc as plsc

B, T, D = 1, 2048, 1024
INTER = 512
E = 8
ROUTE_SCALE = 2.5

BT = 1024
BLK = 256
NB = 23
NPAD = NB * BLK
NTILES = 32
SLOTS_PER_TILE = NPAD // NTILES
DP = D // 2
GCH = 48


def _ffn(xbh, g_ref, u_ref, d_ref):
    dn = (((1,), (1,)), ((), ()))
    g = lax.dot_general(xbh, g_ref[0], dn, preferred_element_type=jnp.float32)
    u = lax.dot_general(xbh, u_ref[0], dn, preferred_element_type=jnp.float32)
    h = (g * (1.0 / (1.0 + jnp.exp(-g)))) * u
    return lax.dot_general(h.astype(jnp.bfloat16), d_ref[0], dn,
                           preferred_element_type=jnp.float32)


def _k1_body(x_ref, gate_ref, sg_ref, su_ref, sd_ref,
             shared_ref, aux_ref, i1_ref, i2_ref, w1_ref, w2_ref,
             r1_ref, r2_ref, cnt_ref, sums_ref, carry_ref):
    t = pl.program_id(0)
    xb = x_ref[:]

    logits = lax.dot_general(
        xb, gate_ref[:], (((1,), (1,)), ((), ())),
        preferred_element_type=jnp.float32) * ROUTE_SCALE
    mx = jnp.max(logits, axis=1, keepdims=True)
    ex = jnp.exp(logits - mx)
    scores = ex / jnp.sum(ex, axis=1, keepdims=True)
    iota8 = lax.broadcasted_iota(jnp.int32, (BT, E), 1)
    m1 = jnp.max(scores, axis=1, keepdims=True)
    i1 = jnp.min(jnp.where(scores == m1, iota8, E), axis=1, keepdims=True)
    masked = jnp.where(iota8 == i1, -jnp.inf, scores)
    m2 = jnp.max(masked, axis=1, keepdims=True)
    i2 = jnp.min(jnp.where(masked == m2, iota8, E), axis=1, keepdims=True)
    s = m1 + m2
    i1_ref[:] = i1
    i2_ref[:] = i2
    w1_ref[:] = m1 / s
    w2_ref[:] = m2 / s

    onehot = ((iota8 == i1) | (iota8 == i2)).astype(jnp.bfloat16)

    @pl.when(t == 0)
    def _():
        sums_ref[:] = jnp.zeros_like(sums_ref)
        carry_ref[:] = jnp.zeros_like(carry_ref)

    rows = lax.broadcasted_iota(jnp.int32, (BT, BT), 0)
    cols = lax.broadcasted_iota(jnp.int32, (BT, BT), 1)
    tri = (cols < rows).astype(jnp.bfloat16)
    excl = lax.dot_general(tri, onehot, (((1,), (0,)), ((), ())),
                           preferred_element_type=jnp.float32)
    excl = excl + carry_ref[0:1, :]
    r1_ref[:] = jnp.sum(jnp.where(iota8 == i1, excl, 0.0), axis=1,
                        keepdims=True)
    r2_ref[:] = jnp.sum(jnp.where(iota8 == i2, excl, 0.0), axis=1,
                        keepdims=True)

    ohf = onehot.astype(jnp.float32)
    carry_ref[:] += jnp.sum(ohf, axis=0, keepdims=True)
    cnt_ref[:] = carry_ref[:]
    sums_ref[0:1, :] += jnp.sum(ohf, axis=0, keepdims=True)
    sums_ref[1:2, :] += jnp.sum(scores, axis=0, keepdims=True)
    aux_ref[:] = (E / (T * T)) * jnp.sum(
        sums_ref[0:1, :] * sums_ref[1:2, :], axis=1, keepdims=True)

    shared_ref[:] = _ffn(xb.astype(jnp.bfloat16), sg_ref, su_ref, sd_ref)


def _run_k1(flat, gate_w, sg, su, sd):
    return pl.pallas_call(
        _k1_body,
        grid=(T // BT,),
        in_specs=[
            pl.BlockSpec((BT, D), lambda t: (t, 0)),
            pl.BlockSpec((E, D), lambda t: (0, 0)),
            pl.BlockSpec((1, INTER, D), lambda t: (0, 0, 0)),
            pl.BlockSpec((1, INTER, D), lambda t: (0, 0, 0)),
            pl.BlockSpec((1, D, INTER), lambda t: (0, 0, 0)),
        ],
        out_specs=[
            pl.BlockSpec((BT, D), lambda t: (t, 0)),
            pl.BlockSpec((1, 1), lambda t: (0, 0)),
            pl.BlockSpec((BT, 1), lambda t: (t, 0)),
            pl.BlockSpec((BT, 1), lambda t: (t, 0)),
            pl.BlockSpec((BT, 1), lambda t: (t, 0)),
            pl.BlockSpec((BT, 1), lambda t: (t, 0)),
            pl.BlockSpec((BT, 1), lambda t: (t, 0)),
            pl.BlockSpec((BT, 1), lambda t: (t, 0)),
            pl.BlockSpec((1, E), lambda t: (0, 0)),
        ],
        out_shape=[
            jax.ShapeDtypeStruct((T, D), jnp.float32),
            jax.ShapeDtypeStruct((1, 1), jnp.float32),
            jax.ShapeDtypeStruct((T, 1), jnp.int32),
            jax.ShapeDtypeStruct((T, 1), jnp.int32),
            jax.ShapeDtypeStruct((T, 1), jnp.float32),
            jax.ShapeDtypeStruct((T, 1), jnp.float32),
            jax.ShapeDtypeStruct((T, 1), jnp.float32),
            jax.ShapeDtypeStruct((T, 1), jnp.float32),
            jax.ShapeDtypeStruct((1, E), jnp.float32),
        ],
        scratch_shapes=[
            pltpu.VMEM((2, E), jnp.float32),
            pltpu.VMEM((1, E), jnp.float32),
        ],
        compiler_params=pltpu.CompilerParams(
            dimension_semantics=("arbitrary",)),
    )(flat, gate_w, sg, su, sd)


def _k2_body(xpk_hbm, i1h, i2h, r1h, r2h, w1h, w2h, cnth,
             xs_out, ws_out, s1_out, s2_out, be_out,
             i1v, i2v, r1v, r2v, w1v, w2v, cntv, bsr, bsb, nbr,
             tokv, wsv, s1v, s2v, besv, rows0, rows1, sem0, sem1):
    cid = lax.axis_index("c")
    sid = lax.axis_index("s")
    wid = sid * 2 + cid
    base = wid * SLOTS_PER_TILE

    pltpu.sync_copy(i1h, i1v)
    pltpu.sync_copy(i2h, i2v)
    pltpu.sync_copy(r1h, r1v)
    pltpu.sync_copy(r2h, r2v)
    pltpu.sync_copy(w1h, w1v)
    pltpu.sync_copy(w2h, w2v)
    pltpu.sync_copy(cnth, cntv)

    cnt = cntv[...]
    lane = lax.iota(jnp.int32, 16)
    nbr[...] = (cnt + (BLK - 1)) >> 8
    startb = jnp.zeros((16,), jnp.int32)
    for j in range(E):
        nbj = plsc.load_gather(nbr, [jnp.full((16,), j, jnp.int32)])
        startb = startb + jnp.where(lane > j, nbj, 0)
    bsr[...] = startb * BLK
    bsb[...] = startb

    for off in [i * 16 for i in range(SLOTS_PER_TILE // 16)] + [
            SLOTS_PER_TILE - 16]:
        tokv[pl.ds(base + off, 16)] = jnp.zeros((16,), jnp.int32)

    def _scatter(i, c):
        sl = pl.ds(i * 16, 16)
        toks = lane + i * 16
        e1 = i1v[sl]
        s1 = plsc.load_gather(bsr, [e1]) + r1v[sl]
        plsc.store_scatter(tokv, [s1], toks)
        plsc.store_scatter(wsv, [s1], w1v[sl])
        s1v[sl] = s1
        e2 = i2v[sl]
        s2 = plsc.load_gather(bsr, [e2]) + r2v[sl]
        plsc.store_scatter(tokv, [s2], toks)
        plsc.store_scatter(wsv, [s2], w2v[sl])
        s2v[sl] = s2
        return c
    lax.fori_loop(0, T // 16, _scatter, 0)

    for v in range(2):
        bidx = lane + v * 16
        acc = jnp.full((16,), -1, jnp.int32)
        for e in range(E):
            se = plsc.load_gather(bsb, [jnp.full((16,), e, jnp.int32)])
            acc = acc + jnp.where(bidx >= se, 1, 0)
        besv[pl.ds(v * 16, 16)] = jnp.minimum(acc, E - 1)

    @pl.when(wid == 1)
    def _():
        pltpu.sync_copy(wsv, ws_out)

    @pl.when(wid == 2)
    def _():
        pltpu.sync_copy(s1v, s1_out)

    @pl.when(wid == 3)
    def _():
        pltpu.sync_copy(s2v, s2_out)

    @pl.when(wid == 4)
    def _():
        pltpu.sync_copy(besv, be_out)

    chunks = ((0, 48), (48, 48), (96, 48), (144, 40))
    bufs = (rows0, rows1)
    sems = (sem0, sem1)
    cps = [None, None]
    for c, (off, sz) in enumerate(chunks):
        b = c & 1
        if cps[b] is not None:
            poff, psz = chunks[c - 2]
            cps[b].wait()
            pltpu.sync_copy(bufs[b].at[pl.ds(0, psz)],
                            xs_out.at[pl.ds(base + poff, psz)])
        idx = tokv.at[pl.ds(base + off, sz)]
        cps[b] = pltpu.async_copy(xpk_hbm.at[idx], bufs[b].at[pl.ds(0, sz)],
                                  sems[b])
    for c in (2, 3):
        b = c & 1
        poff, psz = chunks[c]
        cps[b].wait()
        pltpu.sync_copy(bufs[b].at[pl.ds(0, psz)],
                        xs_out.at[pl.ds(base + poff, psz)])


def _run_k2(xpk, i1f, i2f, r1f, r2f, w1f, w2f, cnt16):
    mesh = plsc.VectorSubcoreMesh(core_axis_name="c", subcore_axis_name="s")
    k = functools.partial(
        pl.kernel, mesh=mesh,
        out_type=[
            jax.ShapeDtypeStruct((NPAD, D), jnp.float32),
            jax.ShapeDtypeStruct((NPAD,), jnp.float32),
            jax.ShapeDtypeStruct((T,), jnp.int32),
            jax.ShapeDtypeStruct((T,), jnp.int32),
            jax.ShapeDtypeStruct((32,), jnp.int32),
        ],
        scratch_types=[
            pltpu.VMEM((T,), jnp.int32),
            pltpu.VMEM((T,), jnp.int32),
            pltpu.VMEM((T,), jnp.int32),
            pltpu.VMEM((T,), jnp.int32),
            pltpu.VMEM((T,), jnp.float32),
            pltpu.VMEM((T,), jnp.float32),
            pltpu.VMEM((16,), jnp.int32),
            pltpu.VMEM((16,), jnp.int32),
            pltpu.VMEM((16,), jnp.int32),
            pltpu.VMEM((16,), jnp.int32),
            pltpu.VMEM((NPAD,), jnp.int32),
            pltpu.VMEM((NPAD,), jnp.float32),
            pltpu.VMEM((T,), jnp.int32),
            pltpu.VMEM((T,), jnp.int32),
            pltpu.VMEM((32,), jnp.int32),
            pltpu.VMEM((GCH, D), jnp.float32),
            pltpu.VMEM((GCH, D), jnp.float32),
            pltpu.SemaphoreType.DMA,
            pltpu.SemaphoreType.DMA,
        ],
        compiler_params=pltpu.CompilerParams(needs_layout_passes=False),
    )(_k2_body)
    return k(xpk, i1f, i2f, r1f, r2f, w1f, w2f, cnt16)


def _k3_body(be_ref, xs_ref, w_ref, rg_ref, ru_ref, rd_ref, ys_ref):
    y = _ffn(xs_ref[:].astype(jnp.bfloat16), rg_ref, ru_ref, rd_ref)
    ys_ref[:] = w_ref[:] * y


def _run_k3(xs, wslot, be, rg, ru, rd):
    grid_spec = pltpu.PrefetchScalarGridSpec(
        num_scalar_prefetch=1,
        grid=(NB,),
        in_specs=[
            pl.BlockSpec((BLK, D), lambda i, be: (i, 0)),
            pl.BlockSpec((BLK, 1), lambda i, be: (i, 0)),
            pl.BlockSpec((1, INTER, D), lambda i, be: (be[i], 0, 0)),
            pl.BlockSpec((1, INTER, D), lambda i, be: (be[i], 0, 0)),
            pl.BlockSpec((1, D, INTER), lambda i, be: (be[i], 0, 0)),
        ],
        out_specs=pl.BlockSpec((BLK, D), lambda i, be: (i, 0)),
    )
    return pl.pallas_call(
        _k3_body,
        grid_spec=grid_spec,
        out_shape=jax.ShapeDtypeStruct((NPAD, D), jnp.float32),
        compiler_params=pltpu.CompilerParams(
            dimension_semantics=("arbitrary",)),
    )(be, xs, wslot, rg, ru, rd)


def _k4_body(shared_hbm, ys_hbm, s1_hbm, s2_hbm, out_hbm,
             iv1, iv2, acc, sem):
    cid = lax.axis_index("c")
    sid = lax.axis_index("s")
    wid = sid * 2 + cid
    base = wid * (T // NTILES)
    n = T // NTILES

    pltpu.sync_copy(s1_hbm.at[pl.ds(base, n)], iv1)
    pltpu.sync_copy(s2_hbm.at[pl.ds(base, n)], iv2)
    pltpu.sync_copy(shared_hbm.at[pl.ds(base, n)], acc)
    pltpu.async_copy(ys_hbm.at[iv1], acc, sem, add=True).wait()
    pltpu.async_copy(ys_hbm.at[iv2], acc, sem, add=True).wait()
    pltpu.sync_copy(acc, out_hbm.at[pl.ds(base, n)])


def _run_k4(shared, ys, s1, s2):
    mesh = plsc.VectorSubcoreMesh(core_axis_name="c", subcore_axis_name="s")
    k = functools.partial(
        pl.kernel, mesh=mesh,
        out_type=jax.ShapeDtypeStruct((T, D), jnp.float32),
        scratch_types=[
            pltpu.VMEM((T // NTILES,), jnp.int32),
            pltpu.VMEM((T // NTILES,), jnp.int32),
            pltpu.VMEM((T // NTILES, D), jnp.float32),
            pltpu.SemaphoreType.DMA,
        ],
        compiler_params=pltpu.CompilerParams(needs_layout_passes=False),
    )(_k4_body)
    return k(shared, ys, s1, s2)


@jax.jit
def kernel(x, gate_w, shared_gate, shared_up, shared_down,
           routed_gate, routed_up, routed_down):
    flat = x.reshape(T, D)
    bf = jnp.bfloat16

    (shared, aux, i1, i2, w1, w2, r1, r2, cnt) = _run_k1(
        flat, gate_w, shared_gate.astype(bf), shared_up.astype(bf),
        shared_down.astype(bf))

    i1f = i1.reshape(T)
    i2f = i2.reshape(T)
    r1f = r1.reshape(T).astype(jnp.int32)
    r2f = r2.reshape(T).astype(jnp.int32)
    w1f = w1.reshape(T)
    w2f = w2.reshape(T)
    cnt16 = jnp.zeros((16,), jnp.int32).at[:E].set(
        cnt.reshape(E).astype(jnp.int32))

    xs, wslot, s1, s2, be = _run_k2(
        flat, i1f, i2f, r1f, r2f, w1f, w2f, cnt16)

    ys = _run_k3(xs, wslot.reshape(NPAD, 1), be,
                 routed_gate.astype(bf), routed_up.astype(bf),
                 routed_down.astype(bf))

    out = _run_k4(shared, ys, s1, s2)
    return out.reshape(B, T, D), aux[0, 0]


# speedup vs baseline: 2.3985x; 1.5171x over previous
import functools

import jax
import jax.numpy as jnp
from jax import lax
from jax.experimental import pallas as pl
from jax.experimental.pallas import tpu as pltpu
from jax.experimental.pallas import tpu_sc as plsc

B, T, D = 1, 2048, 1024
INTER = 512
E = 8
ROUTE_SCALE = 2.5

BT = 1024
BLK = 256
NB = 23
NPAD = NB * BLK
NTILES = 32
NTOK = T // NTILES


def _ffn(xbh, g_ref, u_ref, d_ref):
    dn = (((1,), (1,)), ((), ()))
    g = lax.dot_general(xbh, g_ref[0], dn, preferred_element_type=jnp.float32)
    u = lax.dot_general(xbh, u_ref[0], dn, preferred_element_type=jnp.float32)
    h = (g * (1.0 / (1.0 + jnp.exp(-g)))) * u
    return lax.dot_general(h.astype(jnp.bfloat16), d_ref[0], dn,
                           preferred_element_type=jnp.float32)


def _k1_body(x_ref, gate_ref,
             aux_ref, i1_ref, i2_ref, w1_ref, w2_ref,
             r1_ref, r2_ref, cnt_ref, sums_ref, carry_ref):
    t = pl.program_id(0)
    xb = x_ref[:]

    logits = lax.dot_general(
        xb, gate_ref[:], (((1,), (1,)), ((), ())),
        preferred_element_type=jnp.float32) * ROUTE_SCALE
    mx = jnp.max(logits, axis=1, keepdims=True)
    ex = jnp.exp(logits - mx)
    scores = ex / jnp.sum(ex, axis=1, keepdims=True)
    iota8 = lax.broadcasted_iota(jnp.int32, (BT, E), 1)
    m1 = jnp.max(scores, axis=1, keepdims=True)
    i1 = jnp.min(jnp.where(scores == m1, iota8, E), axis=1, keepdims=True)
    masked = jnp.where(iota8 == i1, -jnp.inf, scores)
    m2 = jnp.max(masked, axis=1, keepdims=True)
    i2 = jnp.min(jnp.where(masked == m2, iota8, E), axis=1, keepdims=True)
    s = m1 + m2
    i1_ref[:] = i1
    i2_ref[:] = i2
    w1_ref[:] = m1 / s
    w2_ref[:] = m2 / s

    onehot = ((iota8 == i1) | (iota8 == i2)).astype(jnp.bfloat16)

    @pl.when(t == 0)
    def _():
        sums_ref[:] = jnp.zeros_like(sums_ref)
        carry_ref[:] = jnp.zeros_like(carry_ref)

    rows = lax.broadcasted_iota(jnp.int32, (BT, BT), 0)
    cols = lax.broadcasted_iota(jnp.int32, (BT, BT), 1)
    tri = (cols < rows).astype(jnp.bfloat16)
    excl = lax.dot_general(tri, onehot, (((1,), (0,)), ((), ())),
                           preferred_element_type=jnp.float32)
    excl = excl + carry_ref[0:1, :]
    r1_ref[:] = jnp.sum(jnp.where(iota8 == i1, excl, 0.0), axis=1,
                        keepdims=True)
    r2_ref[:] = jnp.sum(jnp.where(iota8 == i2, excl, 0.0), axis=1,
                        keepdims=True)

    ohf = onehot.astype(jnp.float32)
    carry_ref[:] += jnp.sum(ohf, axis=0, keepdims=True)
    cnt_ref[:] = carry_ref[:]
    sums_ref[0:1, :] += jnp.sum(ohf, axis=0, keepdims=True)
    sums_ref[1:2, :] += jnp.sum(scores, axis=0, keepdims=True)
    aux_ref[:] = (E / (T * T)) * jnp.sum(
        sums_ref[0:1, :] * sums_ref[1:2, :], axis=1, keepdims=True)


def _run_k1(flat, gate_w):
    return pl.pallas_call(
        _k1_body,
        grid=(T // BT,),
        in_specs=[
            pl.BlockSpec((BT, D), lambda t: (t, 0)),
            pl.BlockSpec((E, D), lambda t: (0, 0)),
        ],
        out_specs=[
            pl.BlockSpec((1, 1), lambda t: (0, 0)),
            pl.BlockSpec((BT, 1), lambda t: (t, 0)),
            pl.BlockSpec((BT, 1), lambda t: (t, 0)),
            pl.BlockSpec((BT, 1), lambda t: (t, 0)),
            pl.BlockSpec((BT, 1), lambda t: (t, 0)),
            pl.BlockSpec((BT, 1), lambda t: (t, 0)),
            pl.BlockSpec((BT, 1), lambda t: (t, 0)),
            pl.BlockSpec((1, E), lambda t: (0, 0)),
        ],
        out_shape=[
            jax.ShapeDtypeStruct((1, 1), jnp.float32),
            jax.ShapeDtypeStruct((T, 1), jnp.int32),
            jax.ShapeDtypeStruct((T, 1), jnp.int32),
            jax.ShapeDtypeStruct((T, 1), jnp.float32),
            jax.ShapeDtypeStruct((T, 1), jnp.float32),
            jax.ShapeDtypeStruct((T, 1), jnp.float32),
            jax.ShapeDtypeStruct((T, 1), jnp.float32),
            jax.ShapeDtypeStruct((1, E), jnp.float32),
        ],
        scratch_shapes=[
            pltpu.VMEM((2, E), jnp.float32),
            pltpu.VMEM((1, E), jnp.float32),
        ],
        compiler_params=pltpu.CompilerParams(
            dimension_semantics=("arbitrary",)),
    )(flat, gate_w)


def _kS_body(x_ref, sg_ref, su_ref, sd_ref, shared_ref):
    shared_ref[:] = _ffn(x_ref[:].astype(jnp.bfloat16), sg_ref, su_ref,
                         sd_ref)


def _run_kS(flat, sg, su, sd):
    return pl.pallas_call(
        _kS_body,
        grid=(T // BT,),
        in_specs=[
            pl.BlockSpec((BT, D), lambda t: (t, 0)),
            pl.BlockSpec((1, INTER, D), lambda t: (0, 0, 0)),
            pl.BlockSpec((1, INTER, D), lambda t: (0, 0, 0)),
            pl.BlockSpec((1, D, INTER), lambda t: (0, 0, 0)),
        ],
        out_specs=pl.BlockSpec((BT, D), lambda t: (t, 0)),
        out_shape=jax.ShapeDtypeStruct((T, D), jnp.float32),
        compiler_params=pltpu.CompilerParams(
            dimension_semantics=("arbitrary",)),
    )(flat, sg, su, sd)


def _k2_body(x_hbm, i1h, i2h, r1h, r2h, w1h, w2h, cnth,
             xs_out, ws_out, s1_out, s2_out, be_out,
             i1v, i2v, r1v, r2v, w1v, w2v, cntv, bsr, bsb, nbr,
             s1v, s2v, besv, xv, sem0, sem1, sem2, sem3):
    cid = lax.axis_index("c")
    sid = lax.axis_index("s")
    wid = sid * 2 + cid
    base = wid * NTOK

    pltpu.sync_copy(i1h.at[pl.ds(base, NTOK)], i1v)
    pltpu.sync_copy(i2h.at[pl.ds(base, NTOK)], i2v)
    pltpu.sync_copy(r1h.at[pl.ds(base, NTOK)], r1v)
    pltpu.sync_copy(r2h.at[pl.ds(base, NTOK)], r2v)
    pltpu.sync_copy(w1h.at[pl.ds(base, NTOK)], w1v)
    pltpu.sync_copy(w2h.at[pl.ds(base, NTOK)], w2v)
    pltpu.sync_copy(cnth, cntv)

    cnt = cntv[...]
    lane = lax.iota(jnp.int32, 16)
    nbr[...] = (cnt + (BLK - 1)) >> 8
    startb = jnp.zeros((16,), jnp.int32)
    for j in range(E):
        nbj = plsc.load_gather(nbr, [jnp.full((16,), j, jnp.int32)])
        startb = startb + jnp.where(lane > j, nbj, 0)
    bsr[...] = startb * BLK
    bsb[...] = startb

    for j in range(NTOK // 16):
        sl = pl.ds(j * 16, 16)
        s1v[sl] = plsc.load_gather(bsr, [i1v[sl]]) + r1v[sl]
        s2v[sl] = plsc.load_gather(bsr, [i2v[sl]]) + r2v[sl]

    pltpu.sync_copy(s1v, s1_out.at[pl.ds(base, NTOK)])
    pltpu.sync_copy(s2v, s2_out.at[pl.ds(base, NTOK)])

    @pl.when(wid == 0)
    def _():
        for v in range(2):
            bidx = lane + v * 16
            acc = jnp.full((16,), -1, jnp.int32)
            for e in range(E):
                se = plsc.load_gather(bsb, [jnp.full((16,), e, jnp.int32)])
                acc = acc + jnp.where(bidx >= se, 1, 0)
            besv[pl.ds(v * 16, 16)] = jnp.minimum(acc, E - 1)
        pltpu.sync_copy(besv, be_out)

    pltpu.sync_copy(x_hbm.at[pl.ds(base, NTOK)], xv)
    c0 = pltpu.async_copy(xv, xs_out.at[s1v], sem0)
    c1 = pltpu.async_copy(xv, xs_out.at[s2v], sem1)
    c2 = pltpu.async_copy(w1v, ws_out.at[s1v], sem2)
    c3 = pltpu.async_copy(w2v, ws_out.at[s2v], sem3)
    c0.wait()
    c1.wait()
    c2.wait()
    c3.wait()


def _run_k2(flat, i1f, i2f, r1f, r2f, w1f, w2f, cnt16):
    mesh = plsc.VectorSubcoreMesh(core_axis_name="c", subcore_axis_name="s")
    k = functools.partial(
        pl.kernel, mesh=mesh,
        out_type=[
            jax.ShapeDtypeStruct((NPAD, D), jnp.float32),
            jax.ShapeDtypeStruct((NPAD,), jnp.float32),
            jax.ShapeDtypeStruct((T,), jnp.int32),
            jax.ShapeDtypeStruct((T,), jnp.int32),
            jax.ShapeDtypeStruct((32,), jnp.int32),
        ],
        scratch_types=[
            pltpu.VMEM((NTOK,), jnp.int32),
            pltpu.VMEM((NTOK,), jnp.int32),
            pltpu.VMEM((NTOK,), jnp.int32),
            pltpu.VMEM((NTOK,), jnp.int32),
            pltpu.VMEM((NTOK,), jnp.float32),
            pltpu.VMEM((NTOK,), jnp.float32),
            pltpu.VMEM((16,), jnp.int32),
            pltpu.VMEM((16,), jnp.int32),
            pltpu.VMEM((16,), jnp.int32),
            pltpu.VMEM((16,), jnp.int32),
            pltpu.VMEM((NTOK,), jnp.int32),
            pltpu.VMEM((NTOK,), jnp.int32),
            pltpu.VMEM((32,), jnp.int32),
            pltpu.VMEM((NTOK, D), jnp.float32),
            pltpu.SemaphoreType.DMA,
            pltpu.SemaphoreType.DMA,
            pltpu.SemaphoreType.DMA,
            pltpu.SemaphoreType.DMA,
        ],
        compiler_params=pltpu.CompilerParams(needs_layout_passes=False),
    )(_k2_body)
    return k(flat, i1f, i2f, r1f, r2f, w1f, w2f, cnt16)


def _k3_body(be_ref, xs_ref, w_ref, rg_ref, ru_ref, rd_ref, ys_ref):
    y = _ffn(xs_ref[:].astype(jnp.bfloat16), rg_ref, ru_ref, rd_ref)
    ys_ref[:] = w_ref[:] * y


def _run_k3(xs, wslot, be, rg, ru, rd):
    grid_spec = pltpu.PrefetchScalarGridSpec(
        num_scalar_prefetch=1,
        grid=(NB,),
        in_specs=[
            pl.BlockSpec((BLK, D), lambda i, be: (i, 0)),
            pl.BlockSpec((BLK, 1), lambda i, be: (i, 0)),
            pl.BlockSpec((1, INTER, D), lambda i, be: (be[i], 0, 0)),
            pl.BlockSpec((1, INTER, D), lambda i, be: (be[i], 0, 0)),
            pl.BlockSpec((1, D, INTER), lambda i, be: (be[i], 0, 0)),
        ],
        out_specs=pl.BlockSpec((BLK, D), lambda i, be: (i, 0)),
    )
    return pl.pallas_call(
        _k3_body,
        grid_spec=grid_spec,
        out_shape=jax.ShapeDtypeStruct((NPAD, D), jnp.float32),
        compiler_params=pltpu.CompilerParams(
            dimension_semantics=("arbitrary",)),
    )(be, xs, wslot, rg, ru, rd)


def _k4_body(shared_hbm, ys_hbm, s1_hbm, s2_hbm, out_hbm,
             iv1, iv2, acc, sem):
    cid = lax.axis_index("c")
    sid = lax.axis_index("s")
    wid = sid * 2 + cid
    base = wid * (T // NTILES)
    n = T // NTILES

    pltpu.sync_copy(s1_hbm.at[pl.ds(base, n)], iv1)
    pltpu.sync_copy(s2_hbm.at[pl.ds(base, n)], iv2)
    pltpu.sync_copy(shared_hbm.at[pl.ds(base, n)], acc)
    pltpu.async_copy(ys_hbm.at[iv1], acc, sem, add=True).wait()
    pltpu.async_copy(ys_hbm.at[iv2], acc, sem, add=True).wait()
    pltpu.sync_copy(acc, out_hbm.at[pl.ds(base, n)])


def _run_k4(shared, ys, s1, s2):
    mesh = plsc.VectorSubcoreMesh(core_axis_name="c", subcore_axis_name="s")
    k = functools.partial(
        pl.kernel, mesh=mesh,
        out_type=jax.ShapeDtypeStruct((T, D), jnp.float32),
        scratch_types=[
            pltpu.VMEM((T // NTILES,), jnp.int32),
            pltpu.VMEM((T // NTILES,), jnp.int32),
            pltpu.VMEM((T // NTILES, D), jnp.float32),
            pltpu.SemaphoreType.DMA,
        ],
        compiler_params=pltpu.CompilerParams(needs_layout_passes=False),
    )(_k4_body)
    return k(shared, ys, s1, s2)


@jax.jit
def kernel(x, gate_w, shared_gate, shared_up, shared_down,
           routed_gate, routed_up, routed_down):
    flat = x.reshape(T, D)
    bf = jnp.bfloat16

    (aux, i1, i2, w1, w2, r1, r2, cnt) = _run_k1(flat, gate_w)

    shared = _run_kS(flat, shared_gate.astype(bf), shared_up.astype(bf),
                     shared_down.astype(bf))

    i1f = i1.reshape(T)
    i2f = i2.reshape(T)
    r1f = r1.reshape(T).astype(jnp.int32)
    r2f = r2.reshape(T).astype(jnp.int32)
    w1f = w1.reshape(T)
    w2f = w2.reshape(T)
    cnt16 = jnp.zeros((16,), jnp.int32).at[:E].set(
        cnt.reshape(E).astype(jnp.int32))

    xs, wslot, s1, s2, be = _run_k2(
        flat, i1f, i2f, r1f, r2f, w1f, w2f, cnt16)

    ys = _run_k3(xs, wslot.reshape(NPAD, 1), be,
                 routed_gate.astype(bf), routed_up.astype(bf),
                 routed_down.astype(bf))

    out = _run_k4(shared, ys, s1, s2)
    return out.reshape(B, T, D), aux[0, 0]
